# baseline (speedup 1.0000x reference)
import functools
import jax
import jax.numpy as jnp
from jax import lax
from jax.experimental import pallas as pl
from jax.experimental.pallas import tpu as pltpu
from jax.experimental.pallas import tpu_sc as plsc

_NC = 2
_NS = 16
_NW = _NC * _NS
_CHUNK = 832
_NBUF = 4
_BLK = 2048


def _transpose_body(t_ref, o_ref):
    t = t_ref[...]
    t = t.reshape(32, _BLK // 4, 4)
    t = jnp.transpose(t, (1, 2, 0))
    o_ref[...] = t.reshape(_BLK // 4, 128)


def kernel(X, table):
    B, F = X.shape
    N = B * F
    V, D = table.shape
    idx = X.reshape(N)

    packed = pl.pallas_call(
        _transpose_body,
        grid=(pl.cdiv(V, _BLK),),
        in_specs=[pl.BlockSpec((D, _BLK), lambda i: (0, i))],
        out_specs=pl.BlockSpec((_BLK // 4, 4 * D), lambda i: (i, 0)),
        out_shape=jax.ShapeDtypeStruct((V // 4, 4 * D), jnp.float32),
    )(table.T)
    table_rows = packed.reshape(V, D)

    b_per_w = N // _NW
    n_chunks = b_per_w // _CHUNK
    n_groups = n_chunks // _NBUF
    assert N % _NW == 0 and b_per_w % (_CHUNK * _NBUF) == 0

    mesh = plsc.VectorSubcoreMesh(core_axis_name="c", subcore_axis_name="s")

    @functools.partial(
        pl.kernel,
        mesh=mesh,
        out_type=jax.ShapeDtypeStruct((N, D), jnp.float32),
        compiler_params=pltpu.CompilerParams(use_tc_tiling_on_sc=False),
        scratch_types=(
            [pltpu.VMEM((b_per_w,), jnp.int32)]
            + [pltpu.VMEM((_CHUNK, D), jnp.float32) for _ in range(_NBUF)]
            + [pltpu.SemaphoreType.DMA for _ in range(2 * _NBUF)]
        ),
    )
    def gather_kernel(table_hbm, idx_hbm, out_hbm, idx_v, *bufs_and_sems):
        rows = bufs_and_sems[:_NBUF]
        gsem = bufs_and_sems[_NBUF:2 * _NBUF]
        osem = bufs_and_sems[2 * _NBUF:]

        wid = lax.axis_index("s") * _NC + lax.axis_index("c")
        base = wid * b_per_w

        pltpu.sync_copy(idx_hbm.at[pl.ds(base, b_per_w)], idx_v)

        def start_gather(c, b):
            pltpu.async_copy(
                table_hbm.at[idx_v.at[pl.ds(c * _CHUNK, _CHUNK)]],
                rows[b], gsem[b])

        def wait_gather(b):
            pltpu.make_async_copy(
                table_hbm.at[idx_v.at[pl.ds(0, _CHUNK)]],
                rows[b], gsem[b]).wait()

        def start_out(c, b):
            pltpu.async_copy(
                rows[b], out_hbm.at[pl.ds(base + c * _CHUNK, _CHUNK)],
                osem[b])

        def wait_out(b):
            pltpu.make_async_copy(
                rows[b], out_hbm.at[pl.ds(base, _CHUNK)], osem[b]).wait()

        for b in range(_NBUF):
            start_gather(b, b)

        @pl.loop(0, n_groups - 1)
        def _(g):
            c0 = g * _NBUF
            for b in range(_NBUF):
                wait_gather(b)
                start_out(c0 + b, b)
            for b in range(_NBUF):
                wait_out(b)
                start_gather(c0 + _NBUF + b, b)

        c0 = (n_groups - 1) * _NBUF
        for b in range(_NBUF):
            wait_gather(b)
            start_out(c0 + b, b)
        for b in range(_NBUF):
            wait_out(b)

    out = gather_kernel(table_rows, idx)
    return out.reshape(B, F, D)


# speedup vs baseline: 3.8014x; 3.8014x over previous
import functools
import jax
import jax.numpy as jnp
from jax import lax
from jax.experimental import pallas as pl
from jax.experimental.pallas import tpu as pltpu
from jax.experimental.pallas import tpu_sc as plsc

_NC = 2
_NS = 16
_NW = _NC * _NS
_CHUNK = 832
_NBUF = 4
_BLK = 2048


def _transpose_body(t_ref, o_ref):
    t = t_ref[...]
    q = _BLK // 4
    o_ref[...] = jnp.concatenate(
        [t[:, k * q:(k + 1) * q].T for k in range(4)], axis=1)


def kernel(X, table):
    B, F = X.shape
    N = B * F
    V, D = table.shape
    idx = X.reshape(N)

    n_blk = pl.cdiv(V, _BLK)
    packed = pl.pallas_call(
        _transpose_body,
        grid=(n_blk,),
        in_specs=[pl.BlockSpec((D, _BLK), lambda i: (0, i))],
        out_specs=pl.BlockSpec((_BLK // 4, 4 * D), lambda i: (i, 0)),
        out_shape=jax.ShapeDtypeStruct((n_blk * _BLK // 4, 4 * D),
                                       jnp.float32),
    )(table.T)
    table_rows = packed.reshape(n_blk * _BLK, D)

    b_per_w = N // _NW
    n_chunks = b_per_w // _CHUNK
    n_groups = n_chunks // _NBUF
    assert N % _NW == 0 and b_per_w % (_CHUNK * _NBUF) == 0

    mesh = plsc.VectorSubcoreMesh(core_axis_name="c", subcore_axis_name="s")

    @functools.partial(
        pl.kernel,
        mesh=mesh,
        out_type=jax.ShapeDtypeStruct((N, D), jnp.float32),
        compiler_params=pltpu.CompilerParams(use_tc_tiling_on_sc=False),
        scratch_types=(
            [pltpu.VMEM((b_per_w,), jnp.int32)]
            + [pltpu.VMEM((_CHUNK, D), jnp.float32) for _ in range(_NBUF)]
            + [pltpu.SemaphoreType.DMA for _ in range(2 * _NBUF)]
        ),
    )
    def gather_kernel(table_hbm, idx_hbm, out_hbm, idx_v, *bufs_and_sems):
        rows = bufs_and_sems[:_NBUF]
        gsem = bufs_and_sems[_NBUF:2 * _NBUF]
        osem = bufs_and_sems[2 * _NBUF:]

        wid = lax.axis_index("s") * _NC + lax.axis_index("c")
        base = wid * b_per_w

        pltpu.sync_copy(idx_hbm.at[pl.ds(base, b_per_w)], idx_v)

        @pl.loop(0, b_per_w // 16)
        def _(j):
            s = idx_v.at[pl.ds(j * 16, 16)][...]
            q = ((s & jnp.int32(~2047))
                 | ((s & jnp.int32(511)) << 2)
                 | ((s & jnp.int32(2047)) >> 9))
            idx_v.at[pl.ds(j * 16, 16)][...] = q

        def start_gather(c, b):
            pltpu.async_copy(
                table_hbm.at[idx_v.at[pl.ds(c * _CHUNK, _CHUNK)]],
                rows[b], gsem[b])

        def wait_gather(b):
            pltpu.make_async_copy(
                table_hbm.at[idx_v.at[pl.ds(0, _CHUNK)]],
                rows[b], gsem[b]).wait()

        def start_out(c, b):
            pltpu.async_copy(
                rows[b], out_hbm.at[pl.ds(base + c * _CHUNK, _CHUNK)],
                osem[b])

        def wait_out(b):
            pltpu.make_async_copy(
                rows[b], out_hbm.at[pl.ds(base, _CHUNK)], osem[b]).wait()

        for b in range(_NBUF):
            start_gather(b, b)

        @pl.loop(0, n_groups - 1)
        def _(g):
            c0 = g * _NBUF
            for b in range(_NBUF):
                wait_gather(b)
                start_out(c0 + b, b)
            for b in range(_NBUF):
                wait_out(b)
                start_gather(c0 + _NBUF + b, b)

        c0 = (n_groups - 1) * _NBUF
        for b in range(_NBUF):
            wait_gather(b)
            start_out(c0 + b, b)
        for b in range(_NBUF):
            wait_out(b)

    out = gather_kernel(table_rows, idx)
    return out.reshape(B, F, D)
